# Initial kernel scaffold; baseline (speedup 1.0000x reference)
import jax
import jax.numpy as jnp
from jax.experimental import pallas as pl


def kernel(pred_disp, coords, prop_E, prop_A, prop_I22, connectivity):
    raise NotImplementedError("write your pallas kernel here")


# speedup vs baseline: 72.8890x; 72.8890x over previous
import functools

import jax
import jax.numpy as jnp
from jax import lax
from jax.experimental import pallas as pl
from jax.experimental.pallas import tpu as pltpu
from jax.experimental.pallas import tpu_sc as plsc

N = 100000
E = 6400000
NC, NS, L = 2, 16, 16
NW = NC * NS
EW = E // NW
K = 800
K3 = 3 * K
NB = EW // K
G = K // L
ROWS_T = 6256
ROWS_LAST = N - (NS - 1) * ROWS_T

_MESH = plsc.VectorSubcoreMesh(
    core_axis_name="c", subcore_axis_name="s", num_cores=NC, num_subcores=NS)

_f32 = jnp.float32
_i32 = jnp.int32


def _rsqrt(d2):
    i = plsc.bitcast(d2, _i32)
    i = jnp.int32(0x5F3759DF) - lax.shift_right_logical(i, 1)
    r = plsc.bitcast(i, _f32)
    hd = 0.5 * d2
    for _ in range(3):
        r = r * (1.5 - hd * r * r)
    return r


@functools.partial(
    pl.kernel,
    out_type=(
        jax.ShapeDtypeStruct((NC, N * 3), _f32),
        jax.ShapeDtypeStruct((E * 3,), _f32),
        jax.ShapeDtypeStruct((E * 3,), _f32),
        jax.ShapeDtypeStruct((E, 6), _f32),
        jax.ShapeDtypeStruct((E, 6), _f32),
        jax.ShapeDtypeStruct((E,), _f32),
        jax.ShapeDtypeStruct((E,), _f32),
        jax.ShapeDtypeStruct((E,), _f32),
        jax.ShapeDtypeStruct((E,), _f32),
        jax.ShapeDtypeStruct((E,), _f32),
        jax.ShapeDtypeStruct((E,), _f32),
        jax.ShapeDtypeStruct((E,), _f32),
        jax.ShapeDtypeStruct((E,), _f32),
    ),
    mesh=_MESH,
    compiler_params=pltpu.CompilerParams(
        needs_layout_passes=False, use_tc_tiling_on_sc=False),
    scratch_types=[
        pltpu.VMEM((K,), _i32),
        pltpu.VMEM((K,), _i32),
        pltpu.VMEM((K3,), _i32),
        pltpu.VMEM((K3,), _i32),
        pltpu.VMEM((K, 8), _f32),
        pltpu.VMEM((K, 8), _f32),
        pltpu.VMEM((K,), _f32),
        pltpu.VMEM((K,), _f32),
        pltpu.VMEM((K,), _f32),
        pltpu.VMEM((K3,), _f32),
        pltpu.VMEM((K3,), _f32),
        pltpu.VMEM((K, 6), _f32),
        pltpu.VMEM((K, 6), _f32),
        pltpu.VMEM((K,), _f32),
        pltpu.VMEM((K,), _f32),
        pltpu.VMEM((K,), _f32),
        pltpu.VMEM((K,), _f32),
        pltpu.VMEM((K,), _f32),
        pltpu.VMEM((K,), _f32),
        pltpu.VMEM((K,), _f32),
        pltpu.VMEM((K,), _f32),
        pltpu.VMEM_SHARED((N * 3,), _f32),
        pltpu.SemaphoreType.DMA,
        pltpu.SemaphoreType.DMA,
        pltpu.SemaphoreType.DMA,
    ],
)
def _beam_sc(tbl, idxA, idxB, pe, pa, pi, zwords,
             o_part, o_fga, o_fgb, o_fl, o_dl, o_ne, o_mm, o_ve, o_m1,
             o_m2, o_l0, o_c, o_s,
             idxA_v, idxB_v, ia3_v, ib3_v, rA_v, rB_v, pe_v, pa_v, pi_v,
             fga_v, fgb_v, fl_v, dl_v,
             ne_v, mm_v, ve_v, m1_v, m2_v, l0_v, c_v, s_v,
             acc, semA, semB, semO):
    cid = lax.axis_index("c")
    sid = lax.axis_index("s")
    wid = cid * NS + sid
    w0 = sid * (ROWS_T * 3)

    @pl.when(sid < NS - 1)
    def _():
        pltpu.sync_copy(zwords.at[pl.ds(0, ROWS_T * 3)],
                        acc.at[pl.ds(w0, ROWS_T * 3)])

    @pl.when(sid == NS - 1)
    def _():
        pltpu.sync_copy(zwords.at[pl.ds(0, ROWS_LAST * 3)],
                        acc.at[pl.ds((NS - 1) * ROWS_T * 3, ROWS_LAST * 3)])

    plsc.subcore_barrier()

    iota = lax.iota(_i32, L)
    cols = [jnp.full((L,), j, _i32) for j in range(6)]

    @pl.loop(0, NB)
    def _block(bi):
        base = wid * EW + bi * K
        pltpu.sync_copy(idxA.at[pl.ds(base, K)], idxA_v)
        pltpu.sync_copy(idxB.at[pl.ds(base, K)], idxB_v)
        ga = pltpu.async_copy(tbl.at[idxA_v], rA_v, semA)
        gb = pltpu.async_copy(tbl.at[idxB_v], rB_v, semB)
        pltpu.sync_copy(pe.at[pl.ds(base, K)], pe_v)
        pltpu.sync_copy(pa.at[pl.ds(base, K)], pa_v)
        pltpu.sync_copy(pi.at[pl.ds(base, K)], pi_v)
        ga.wait()
        gb.wait()

        @pl.loop(0, G)
        def _grp(g):
            off = g * L
            rid = off + iota
            rid3 = rid * 3
            uxA = plsc.load_gather(rA_v, [rid, cols[0]])
            uzA = plsc.load_gather(rA_v, [rid, cols[1]])
            thA = plsc.load_gather(rA_v, [rid, cols[2]])
            xA = plsc.load_gather(rA_v, [rid, cols[3]])
            zA = plsc.load_gather(rA_v, [rid, cols[4]])
            uxB = plsc.load_gather(rB_v, [rid, cols[0]])
            uzB = plsc.load_gather(rB_v, [rid, cols[1]])
            thB = plsc.load_gather(rB_v, [rid, cols[2]])
            xB = plsc.load_gather(rB_v, [rid, cols[3]])
            zB = plsc.load_gather(rB_v, [rid, cols[4]])
            pE = pe_v[pl.ds(off, L)]
            pA = pa_v[pl.ds(off, L)]
            pI = pi_v[pl.ds(off, L)]
            na = idxA_v[pl.ds(off, L)]
            nb = idxB_v[pl.ds(off, L)]

            dx = xB - xA
            dz = zB - zA
            d2 = dx * dx + dz * dz
            r = _rsqrt(d2)
            cc = dx * r
            ss = dz * r
            l0 = d2 * r
            r2 = r * r
            EA = pE * pA
            EI = pE * pI
            EAr = EA * r
            EIr = EI * r
            EIr2 = EI * r2
            EIr3 = EIr * r2

            ua = cc * uxA + ss * uzA
            wa = cc * uzA - ss * uxA
            ub = cc * uxB + ss * uzB
            wb = cc * uzB - ss * uxB
            du = ua - ub
            dw = wa - wb
            tsum = thA + thB

            f0 = EAr * du
            f1 = 12.0 * (EIr3 * dw) + 6.0 * (EIr2 * tsum)
            cdw = 6.0 * (EIr2 * dw)
            f2 = cdw + EIr * (4.0 * thA + 2.0 * thB)
            f5 = cdw + EIr * (2.0 * thA + 4.0 * thB)
            f3 = -f0
            f4 = -f1
            FxA = cc * f0 - ss * f1
            FzA = ss * f0 + cc * f1
            mm = (f5 - f2) * 0.5

            a3 = na * 3
            b3 = nb * 3
            plsc.store_scatter(fga_v, [rid3], FxA)
            plsc.store_scatter(fga_v, [rid3 + 1], FzA)
            plsc.store_scatter(fga_v, [rid3 + 2], f2)
            plsc.store_scatter(fgb_v, [rid3], -FxA)
            plsc.store_scatter(fgb_v, [rid3 + 1], -FzA)
            plsc.store_scatter(fgb_v, [rid3 + 2], f5)
            plsc.store_scatter(ia3_v, [rid3], a3)
            plsc.store_scatter(ia3_v, [rid3 + 1], a3 + 1)
            plsc.store_scatter(ia3_v, [rid3 + 2], a3 + 2)
            plsc.store_scatter(ib3_v, [rid3], b3)
            plsc.store_scatter(ib3_v, [rid3 + 1], b3 + 1)
            plsc.store_scatter(ib3_v, [rid3 + 2], b3 + 2)
            plsc.store_scatter(fl_v, [rid, cols[0]], f0)
            plsc.store_scatter(fl_v, [rid, cols[1]], f1)
            plsc.store_scatter(fl_v, [rid, cols[2]], f2)
            plsc.store_scatter(fl_v, [rid, cols[3]], f3)
            plsc.store_scatter(fl_v, [rid, cols[4]], f4)
            plsc.store_scatter(fl_v, [rid, cols[5]], f5)
            plsc.store_scatter(dl_v, [rid, cols[0]], ua)
            plsc.store_scatter(dl_v, [rid, cols[1]], wa)
            plsc.store_scatter(dl_v, [rid, cols[2]], thA)
            plsc.store_scatter(dl_v, [rid, cols[3]], ub)
            plsc.store_scatter(dl_v, [rid, cols[4]], wb)
            plsc.store_scatter(dl_v, [rid, cols[5]], thB)
            ne_v[pl.ds(off, L)] = f3
            mm_v[pl.ds(off, L)] = mm
            ve_v[pl.ds(off, L)] = f4
            m1_v[pl.ds(off, L)] = f2
            m2_v[pl.ds(off, L)] = f5
            l0_v[pl.ds(off, L)] = l0
            c_v[pl.ds(off, L)] = cc
            s_v[pl.ds(off, L)] = ss

        outs = [
            pltpu.async_copy(fga_v, o_fga.at[pl.ds(base * 3, K3)], semO),
            pltpu.async_copy(fgb_v, o_fgb.at[pl.ds(base * 3, K3)], semO),
            pltpu.async_copy(fl_v, o_fl.at[pl.ds(base, K)], semO),
            pltpu.async_copy(dl_v, o_dl.at[pl.ds(base, K)], semO),
            pltpu.async_copy(ne_v, o_ne.at[pl.ds(base, K)], semO),
            pltpu.async_copy(mm_v, o_mm.at[pl.ds(base, K)], semO),
            pltpu.async_copy(ve_v, o_ve.at[pl.ds(base, K)], semO),
            pltpu.async_copy(m1_v, o_m1.at[pl.ds(base, K)], semO),
            pltpu.async_copy(m2_v, o_m2.at[pl.ds(base, K)], semO),
            pltpu.async_copy(l0_v, o_l0.at[pl.ds(base, K)], semO),
            pltpu.async_copy(c_v, o_c.at[pl.ds(base, K)], semO),
            pltpu.async_copy(s_v, o_s.at[pl.ds(base, K)], semO),
        ]
        pltpu.sync_copy(fga_v, acc.at[ia3_v], add=True)
        pltpu.sync_copy(fgb_v, acc.at[ib3_v], add=True)
        for d in outs:
            d.wait()

    plsc.subcore_barrier()

    @pl.when(sid < NS - 1)
    def _():
        pltpu.sync_copy(acc.at[pl.ds(w0, ROWS_T * 3)],
                        o_part.at[cid, pl.ds(w0, ROWS_T * 3)])

    @pl.when(sid == NS - 1)
    def _():
        pltpu.sync_copy(
            acc.at[pl.ds((NS - 1) * ROWS_T * 3, ROWS_LAST * 3)],
            o_part.at[cid, pl.ds((NS - 1) * ROWS_T * 3, ROWS_LAST * 3)])


def kernel(pred_disp, coords, prop_E, prop_A, prop_I22, connectivity):
    tbl = jnp.concatenate(
        [pred_disp, coords[:, 0:1], coords[:, 2:3],
         jnp.zeros((N, 3), _f32)], axis=1)
    idxA = connectivity[:, 0].astype(_i32)
    idxB = connectivity[:, 1].astype(_i32)
    zwords = jnp.zeros((ROWS_T * 3,), _f32)
    (part, fga, fgb, fl, dl, ne, mm, ve, m1, m2, l0, c, s) = _beam_sc(
        tbl, idxA, idxB, prop_E, prop_A, prop_I22, zwords)
    nodal = (part[0] + part[1]).reshape(N, 3)
    return (nodal, fga.reshape(E, 3), fgb.reshape(E, 3), fl, dl,
            ne, mm, ve, m1, m2, l0, c, s)
